# baseline (speedup 1.0000x reference)
import jax
import jax.numpy as jnp
from jax import lax
from jax.experimental import pallas as pl
from jax.experimental.pallas import tpu as pltpu
from jax.experimental.pallas import tpu_sc as plsc

EMB = 32
BATCH = 16384
NC = 2
NS = 16
NW = NC * NS
BPW = BATCH // NW

_T_BLOCK = 8192
_MLP_BLOCK = 2048


def _repack_body(t_ref, o_ref):
    x = t_ref[...]
    c = x.shape[1]
    x = x.reshape(32, c // 4, 4)
    x = x.transpose(1, 2, 0)
    o_ref[...] = x.reshape(c // 4, 128)


def _repack(table):
    v = table.shape[0]
    t_t = table.T
    if v <= _T_BLOCK:
        in_spec = pl.BlockSpec((EMB, v), lambda i: (0, 0))
        out_spec = pl.BlockSpec((v // 4, 128), lambda i: (0, 0))
        grid = (1,)
    else:
        in_spec = pl.BlockSpec((EMB, _T_BLOCK), lambda i: (0, i))
        out_spec = pl.BlockSpec((_T_BLOCK // 4, 128), lambda i: (i, 0))
        grid = (pl.cdiv(v, _T_BLOCK),)
    return pl.pallas_call(
        _repack_body,
        grid=grid,
        in_specs=[in_spec],
        out_specs=out_spec,
        out_shape=jax.ShapeDtypeStruct((v // 4, 128), jnp.float32),
    )(t_t)


def _sc_gather_body(gq_hbm, aq_hbm, rq_hbm, gt_hbm, at_hbm, rt_hbm,
                    go_hbm, ao_hbm, ro_hbm,
                    idx_v, rows_v, sem):
    wid = lax.axis_index("s") * NC + lax.axis_index("c")
    base = wid * BPW
    for q_hbm, tab_hbm, out_hbm in ((gq_hbm, gt_hbm, go_hbm),
                                    (aq_hbm, at_hbm, ao_hbm),
                                    (rq_hbm, rt_hbm, ro_hbm)):
        pltpu.sync_copy(q_hbm.at[pl.ds(base, BPW)], idx_v)
        pltpu.async_copy(tab_hbm.at[idx_v], rows_v, sem).wait()
        pltpu.sync_copy(rows_v, out_hbm.at[pl.ds(base, BPW)])


_ROWS_OUT = jax.ShapeDtypeStruct((BATCH, 128), jnp.float32)

_sc_gather = pl.kernel(
    _sc_gather_body,
    out_type=[_ROWS_OUT, _ROWS_OUT, _ROWS_OUT],
    mesh=plsc.VectorSubcoreMesh(core_axis_name="c", subcore_axis_name="s"),
    scratch_types=[
        pltpu.VMEM((BPW,), jnp.int32),
        pltpu.VMEM((BPW, 128), jnp.float32),
        pltpu.SemaphoreType.DMA,
    ],
    compiler_params=pltpu.CompilerParams(use_tc_tiling_on_sc=False),
)


def _mlp_body(g_ref, a_ref, r_ref, gi_ref, ai_ref, ri_ref,
              w1_ref, b1_ref, w2_ref, b2_ref, w3_ref, b3_ref, o_ref):
    h = jnp.broadcast_to(b1_ref[...], (g_ref.shape[0], 64))
    for t, (x_ref, i_ref) in enumerate(((g_ref, gi_ref),
                                        (a_ref, ai_ref),
                                        (r_ref, ri_ref))):
        x = x_ref[...]
        p_row = (i_ref[...] & 3).reshape(-1, 1)
        for p in range(4):
            xp = jnp.dot(x, w1_ref[t * 4 + p],
                         preferred_element_type=jnp.float32)
            h = h + jnp.where(p_row == p, xp, 0.0)
    h = jnp.maximum(h, 0.0)
    h = jnp.maximum(
        jnp.dot(h, w2_ref[...], preferred_element_type=jnp.float32)
        + b2_ref[...], 0.0)
    o_ref[...] = (jnp.dot(h, w3_ref[...], preferred_element_type=jnp.float32)
                  + b3_ref[...])


def _mlp(g, a, r, gi, ai, ri, w1_slabs, b1, w2, b2, w3, b3):
    n_blocks = BATCH // _MLP_BLOCK
    emb_spec = pl.BlockSpec((_MLP_BLOCK, 128), lambda i: (i, 0))
    idx_spec = pl.BlockSpec((_MLP_BLOCK,), lambda i: (i,))
    whole = lambda arr: pl.BlockSpec(arr.shape, lambda i: (0,) * arr.ndim)
    return pl.pallas_call(
        _mlp_body,
        grid=(n_blocks,),
        in_specs=[emb_spec, emb_spec, emb_spec,
                  idx_spec, idx_spec, idx_spec,
                  whole(w1_slabs), whole(b1),
                  whole(w2), whole(b2), whole(w3), whole(b3)],
        out_specs=pl.BlockSpec((_MLP_BLOCK, EMB), lambda i: (i, 0)),
        out_shape=jax.ShapeDtypeStruct((BATCH, EMB), jnp.float32),
    )(g, a, r, gi, ai, ri, w1_slabs, b1, w2, b2, w3, b3)


def kernel(genre_id, author_id, artist_id,
           genre_table, author_table, artist_table,
           W1, b1, W2, b2, W3, b3):
    genre_pk = _repack(genre_table)
    author_pk = _repack(author_table)
    artist_pk = _repack(artist_table)

    gq = genre_id >> 2
    aq = author_id >> 2
    rq = artist_id >> 2
    g, a, r = _sc_gather(gq, aq, rq, genre_pk, author_pk, artist_pk)

    slabs = []
    for t in range(3):
        w1t = W1[t * EMB:(t + 1) * EMB]
        for p in range(4):
            slabs.append(jnp.pad(w1t, ((32 * p, 96 - 32 * p), (0, 0))))
    w1_slabs = jnp.stack(slabs)

    return _mlp(g, a, r, genre_id, author_id, artist_id,
                w1_slabs, b1.reshape(1, -1), W2, b2.reshape(1, -1),
                W3, b3.reshape(1, -1))


# speedup vs baseline: 10.1915x; 10.1915x over previous
import jax
import jax.numpy as jnp
from jax import lax
from jax.experimental import pallas as pl
from jax.experimental.pallas import tpu as pltpu
from jax.experimental.pallas import tpu_sc as plsc

EMB = 32
BATCH = 16384
NC = 2
NS = 16
NW = NC * NS
BPW = BATCH // NW

_MLP_BLOCK = 2048


def _repack_body(t_ref, eye_ref, o_ref):
    x = t_ref[...]
    c = x.shape[1] // 4
    cat = jnp.concatenate([x[:, p * c:(p + 1) * c] for p in range(4)],
                          axis=0)
    o_ref[...] = lax.dot_general(
        cat, eye_ref[...], (((0,), (0,)), ((), ())),
        precision=lax.Precision.HIGHEST,
        preferred_element_type=jnp.float32)


def _repack(table, c):
    v = table.shape[0]
    t_t = table.T
    n_blocks = pl.cdiv(v, 4 * c)
    rows = n_blocks * c
    eye = jnp.eye(128, dtype=jnp.float32)
    return pl.pallas_call(
        _repack_body,
        grid=(n_blocks,),
        in_specs=[pl.BlockSpec((EMB, 4 * c), lambda i: (0, i)),
                  pl.BlockSpec((128, 128), lambda i: (0, 0))],
        out_specs=pl.BlockSpec((c, 128), lambda i: (i, 0)),
        out_shape=jax.ShapeDtypeStruct((rows, 128), jnp.float32),
        compiler_params=pltpu.CompilerParams(
            fuse_transposed_lhs_in_matmul=True),
    )(t_t, eye)


def _sc_gather_body(gq_hbm, aq_hbm, rq_hbm, gt_hbm, at_hbm, rt_hbm,
                    go_hbm, ao_hbm, ro_hbm,
                    idx_v, rows_v, sem):
    wid = lax.axis_index("s") * NC + lax.axis_index("c")
    base = wid * BPW
    for q_hbm, tab_hbm, out_hbm in ((gq_hbm, gt_hbm, go_hbm),
                                    (aq_hbm, at_hbm, ao_hbm),
                                    (rq_hbm, rt_hbm, ro_hbm)):
        pltpu.sync_copy(q_hbm.at[pl.ds(base, BPW)], idx_v)
        pltpu.async_copy(tab_hbm.at[idx_v], rows_v, sem).wait()
        pltpu.sync_copy(rows_v, out_hbm.at[pl.ds(base, BPW)])


_ROWS_OUT = jax.ShapeDtypeStruct((BATCH, 128), jnp.float32)


def _make_sc_gather(g_rows, a_rows, r_rows):
    return pl.kernel(
        _sc_gather_body,
        out_type=[_ROWS_OUT, _ROWS_OUT, _ROWS_OUT],
        mesh=plsc.VectorSubcoreMesh(core_axis_name="c", subcore_axis_name="s"),
        scratch_types=[
            pltpu.VMEM((BPW,), jnp.int32),
            pltpu.VMEM((BPW, 128), jnp.float32),
            pltpu.SemaphoreType.DMA,
        ],
        compiler_params=pltpu.CompilerParams(use_tc_tiling_on_sc=False),
    )


def _mlp_body(g_ref, a_ref, r_ref, gp_ref, ap_ref, rp_ref,
              w1_ref, b1_ref, w2_ref, b2_ref, w3_ref, b3_ref, o_ref):
    h = jnp.broadcast_to(b1_ref[...], (g_ref.shape[0], 64))
    for t, (x_ref, p_ref) in enumerate(((g_ref, gp_ref),
                                        (a_ref, ap_ref),
                                        (r_ref, rp_ref))):
        x = x_ref[...]
        p_row = p_ref[...].reshape(-1, 1)
        for p in range(4):
            xp = jnp.dot(x, w1_ref[t * 4 + p],
                         preferred_element_type=jnp.float32)
            h = h + jnp.where(p_row == p, xp, 0.0)
    h = jnp.maximum(h, 0.0)
    h = jnp.maximum(
        jnp.dot(h, w2_ref[...], preferred_element_type=jnp.float32)
        + b2_ref[...], 0.0)
    o_ref[...] = (jnp.dot(h, w3_ref[...], preferred_element_type=jnp.float32)
                  + b3_ref[...])


def _mlp(g, a, r, gp, ap, rp, w1_slabs, b1, w2, b2, w3, b3):
    n_blocks = BATCH // _MLP_BLOCK
    emb_spec = pl.BlockSpec((_MLP_BLOCK, 128), lambda i: (i, 0))
    idx_spec = pl.BlockSpec((_MLP_BLOCK,), lambda i: (i,))
    whole = lambda arr: pl.BlockSpec(arr.shape, lambda i: (0,) * arr.ndim)
    return pl.pallas_call(
        _mlp_body,
        grid=(n_blocks,),
        in_specs=[emb_spec, emb_spec, emb_spec,
                  idx_spec, idx_spec, idx_spec,
                  whole(w1_slabs), whole(b1),
                  whole(w2), whole(b2), whole(w3), whole(b3)],
        out_specs=pl.BlockSpec((_MLP_BLOCK, EMB), lambda i: (i, 0)),
        out_shape=jax.ShapeDtypeStruct((BATCH, EMB), jnp.float32),
    )(g, a, r, gp, ap, rp, w1_slabs, b1, w2, b2, w3, b3)


def _qp(idx, c):
    q = (idx // (4 * c)) * c + (idx % c)
    p = (idx // c) % 4
    return q, p


def kernel(genre_id, author_id, artist_id,
           genre_table, author_table, artist_table,
           W1, b1, W2, b2, W3, b3):
    c_g, c_a, c_r = 256, 2048, 2048
    genre_pk = _repack(genre_table, c_g)
    author_pk = _repack(author_table, c_a)
    artist_pk = _repack(artist_table, c_r)

    gq, gp = _qp(genre_id, c_g)
    aq, ap = _qp(author_id, c_a)
    rq, rp = _qp(artist_id, c_r)

    sc_gather = _make_sc_gather(genre_pk.shape[0], author_pk.shape[0],
                                artist_pk.shape[0])
    g, a, r = sc_gather(gq, aq, rq, genre_pk, author_pk, artist_pk)

    slabs = []
    for t in range(3):
        w1t = W1[t * EMB:(t + 1) * EMB]
        for p in range(4):
            slabs.append(jnp.pad(w1t, ((32 * p, 96 - 32 * p), (0, 0))))
    w1_slabs = jnp.stack(slabs)

    return _mlp(g, a, r, gp, ap, rp,
                w1_slabs, b1.reshape(1, -1), W2, b2.reshape(1, -1),
                W3, b3.reshape(1, -1))
